# baseline (speedup 1.0000x reference)
import functools

import jax
import jax.numpy as jnp
from jax import lax
from jax.experimental import pallas as pl
from jax.experimental.pallas import tpu as pltpu
from jax.experimental.pallas import tpu_sc as plsc

N_R = 10000
NNZ = 320000
U = 128
EPS = 1e-10
NPAD = 10240
NHI = NPAD // 128
NBATCH = NNZ // 128

_mesh = plsc.VectorSubcoreMesh(core_axis_name="c", subcore_axis_name="s",
                               num_cores=2, num_subcores=16)


@functools.partial(
    pl.kernel,
    out_type=jax.ShapeDtypeStruct((2, NPAD, U), jnp.float32),
    mesh=_mesh,
    scratch_types=[
        pltpu.VMEM((2, 128, U), jnp.float32),
        pltpu.VMEM((2, 128), jnp.int32),
        pltpu.VMEM_SHARED((NPAD, U), jnp.float32),
        pltpu.SemaphoreType.DMA,
        pltpu.SemaphoreType.DMA,
    ],
)
def _seg_sums(values, idx2, acc_out, val_b, idx_b, acc_s, sl0, sl1):
    c = lax.axis_index("c")
    s = lax.axis_index("s")
    z16 = jnp.zeros((16,), jnp.float32)

    def _zero_val(r, _):
        for k in range(U // 16):
            val_b[0, r, pl.ds(k * 16, 16)] = z16
        return 0
    lax.fori_loop(0, 128, _zero_val, 0)

    for b in range(5):
        pltpu.sync_copy(val_b.at[0], acc_s.at[pl.ds(s * 640 + b * 128, 128), :])

    plsc.subcore_barrier()

    base = 156 * s + jnp.minimum(s, 4)
    sems = (sl0, sl1)

    def _b0(j):
        return jnp.minimum(base + j, NBATCH - 1) * 128

    def _issue(j, p):
        b0 = _b0(j)
        pltpu.async_copy(values.at[pl.ds(b0, 128), :], val_b.at[p], sems[p])
        pltpu.async_copy(idx2.at[c, pl.ds(b0, 128)], idx_b.at[p], sems[p])

    def _wait(j, p):
        b0 = _b0(j)
        pltpu.make_async_copy(values.at[pl.ds(b0, 128), :], val_b.at[p],
                              sems[p]).wait()
        pltpu.make_async_copy(idx2.at[c, pl.ds(b0, 128)], idx_b.at[p],
                              sems[p]).wait()

    def _scatter(p):
        pltpu.sync_copy(val_b.at[p], acc_s.at[idx_b.at[p]], add=True)

    _issue(0, 0)

    def _acc_body(t, _):
        j0 = 2 * t
        _issue(j0 + 1, 1)
        _wait(j0, 0)
        _scatter(0)
        _issue(j0 + 2, 0)
        _wait(j0 + 1, 1)
        _scatter(1)
        return 0
    lax.fori_loop(0, 78, _acc_body, 0)

    _wait(156, 0)

    @pl.when(s < 4)
    def _():
        _scatter(0)

    plsc.subcore_barrier()

    pltpu.sync_copy(acc_s.at[pl.ds(s * 640, 640), :],
                    acc_out.at[c, pl.ds(s * 640, 640), :])


def _self_mm_body(x_ref, w_ref, i2_ref, o_ref, s_ref, cr_ref, cc_ref):
    x = x_ref[...]
    o_ref[...] = jnp.dot(x, w_ref[...], preferred_element_type=jnp.float32)

    @pl.when(pl.program_id(0) == 0)
    def _():
        s_ref[...] = jnp.zeros_like(s_ref)
        cr_ref[...] = jnp.zeros_like(cr_ref)
        cc_ref[...] = jnp.zeros_like(cc_ref)

    s_ref[...] += jnp.sum(x, axis=0, keepdims=True)

    i2 = i2_ref[...]
    dn = (((0,), (0,)), ((), ()))
    for axis, ref in ((0, cr_ref), (1, cc_ref)):
        idx = i2[axis]
        oh_hi = (lax.broadcasted_iota(jnp.int32, (_MBLK, NHI), 1)
                 == (idx >> 7)[:, None]).astype(jnp.bfloat16)
        oh_lo = (lax.broadcasted_iota(jnp.int32, (_MBLK, U), 1)
                 == (idx & 127)[:, None]).astype(jnp.bfloat16)
        ref[...] += lax.dot_general(oh_hi, oh_lo, dn,
                                    preferred_element_type=jnp.float32)


_MBLK = 1280


def _self_mm(values, theta_self, idx2):
    return pl.pallas_call(
        _self_mm_body,
        grid=(NNZ // _MBLK,),
        in_specs=[
            pl.BlockSpec((_MBLK, U), lambda i: (i, 0)),
            pl.BlockSpec((U, U), lambda i: (0, 0)),
            pl.BlockSpec((2, _MBLK), lambda i: (0, i)),
        ],
        out_specs=[
            pl.BlockSpec((_MBLK, U), lambda i: (i, 0)),
            pl.BlockSpec((1, U), lambda i: (0, 0)),
            pl.BlockSpec((NHI, U), lambda i: (0, 0)),
            pl.BlockSpec((NHI, U), lambda i: (0, 0)),
        ],
        out_shape=[
            jax.ShapeDtypeStruct((NNZ, U), jnp.float32),
            jax.ShapeDtypeStruct((1, U), jnp.float32),
            jax.ShapeDtypeStruct((NHI, U), jnp.float32),
            jax.ShapeDtypeStruct((NHI, U), jnp.float32),
        ],
    )(values, theta_self, idx2)


def _marg_mm_body(a0, a1, cc, thr, thc, tha, vs, brow, bcol):
    bias = jnp.dot(vs[...], tha[...], preferred_element_type=jnp.float32)
    bias = bias * (0.5 / NNZ)
    cnt2 = cc[...]
    m0 = a0[0] / (cnt2[0][:, None] + EPS)
    m1 = a1[0] / (cnt2[1][:, None] + EPS)
    brow[...] = jnp.dot(m0, thr[...], preferred_element_type=jnp.float32) + bias
    bcol[...] = jnp.dot(m1, thc[...], preferred_element_type=jnp.float32) + bias


_BBLK = 1280


def _marg_mm(acc, cnt, theta_row, theta_col, theta_all, vsum):
    return pl.pallas_call(
        _marg_mm_body,
        grid=(NPAD // _BBLK,),
        in_specs=[
            pl.BlockSpec((1, _BBLK, U), lambda i: (0, i, 0)),
            pl.BlockSpec((1, _BBLK, U), lambda i: (1, i, 0)),
            pl.BlockSpec((2, _BBLK), lambda i: (0, i)),
            pl.BlockSpec((U, U), lambda i: (0, 0)),
            pl.BlockSpec((U, U), lambda i: (0, 0)),
            pl.BlockSpec((U, U), lambda i: (0, 0)),
            pl.BlockSpec((1, U), lambda i: (0, 0)),
        ],
        out_specs=[
            pl.BlockSpec((_BBLK, U), lambda i: (i, 0)),
            pl.BlockSpec((_BBLK, U), lambda i: (i, 0)),
        ],
        out_shape=[
            jax.ShapeDtypeStruct((NPAD, U), jnp.float32),
            jax.ShapeDtypeStruct((NPAD, U), jnp.float32),
        ],
    )(acc, acc, cnt, theta_row, theta_col, theta_all, vsum)


@functools.partial(
    pl.kernel,
    out_type=jax.ShapeDtypeStruct((NNZ, U), jnp.float32),
    mesh=_mesh,
    scratch_types=[
        pltpu.VMEM((2, 128, U), jnp.float32),
        pltpu.VMEM((2, 128, U), jnp.float32),
        pltpu.VMEM((2, 128, U), jnp.float32),
        pltpu.VMEM((2, 128), jnp.int32),
        pltpu.VMEM((2, 128), jnp.int32),
        pltpu.SemaphoreType.DMA,
        pltpu.SemaphoreType.DMA,
        pltpu.SemaphoreType.DMA,
        pltpu.SemaphoreType.DMA,
    ],
)
def _gather_finish(self_out, idx2, brow, bcol, out,
                   buf, g1, g2, idxr, idxc, si0, si1, sg0, sg1):
    c = lax.axis_index("c")
    s = lax.axis_index("s")
    w = s * 2 + c
    base = 78 * w + jnp.minimum(w, 4)
    zero16 = jnp.zeros((16,), jnp.float32)
    sems_i = (si0, si1)
    sems_g = (sg0, sg1)

    def _b0(j):
        return jnp.minimum(base + j, NBATCH - 1) * 128

    def _issue_loads(j, p):
        b0 = _b0(j)
        pltpu.async_copy(idx2.at[0, pl.ds(b0, 128)], idxr.at[p], sems_i[p])
        pltpu.async_copy(idx2.at[1, pl.ds(b0, 128)], idxc.at[p], sems_i[p])
        pltpu.async_copy(self_out.at[pl.ds(b0, 128), :], buf.at[p], sems_g[p])

    def _issue_gathers(j, p):
        b0 = _b0(j)
        pltpu.make_async_copy(idx2.at[0, pl.ds(b0, 128)], idxr.at[p],
                              sems_i[p]).wait()
        pltpu.make_async_copy(idx2.at[1, pl.ds(b0, 128)], idxc.at[p],
                              sems_i[p]).wait()
        pltpu.async_copy(brow.at[idxr.at[p]], g1.at[p], sems_g[p])
        pltpu.async_copy(bcol.at[idxc.at[p]], g2.at[p], sems_g[p])

    def _compute_store(j, p):
        b0 = _b0(j)
        pltpu.make_async_copy(self_out.at[pl.ds(b0, 128), :], buf.at[p],
                              sems_g[p]).wait()
        pltpu.make_async_copy(brow.at[idxr.at[p]], g1.at[p], sems_g[p]).wait()
        pltpu.make_async_copy(bcol.at[idxc.at[p]], g2.at[p], sems_g[p]).wait()

        def _relu(t, _):
            for rr in range(4):
                r = 4 * t + rr
                for k in range(U // 16):
                    sl = pl.ds(k * 16, 16)
                    v = buf[p, r, sl] + g1[p, r, sl] + g2[p, r, sl]
                    buf[p, r, sl] = jnp.maximum(v, zero16)
            return 0
        lax.fori_loop(0, 32, _relu, 0)
        pltpu.sync_copy(buf.at[p], out.at[pl.ds(b0, 128), :])

    _issue_loads(0, 0)
    _issue_gathers(0, 0)

    def _body(t, _):
        j0 = 2 * t
        _issue_loads(j0 + 1, 1)
        _issue_gathers(j0 + 1, 1)
        _compute_store(j0, 0)
        _issue_loads(j0 + 2, 0)
        _issue_gathers(j0 + 2, 0)
        _compute_store(j0 + 1, 1)
        return 0
    lax.fori_loop(0, 40, _body, 0)

    _compute_store(80, 0)


def kernel(values, indices, theta_self, theta_row, theta_col, theta_all):
    idx2 = jnp.transpose(indices.astype(jnp.int32))
    self_out, vsum, cnt_r, cnt_c = _self_mm(values, theta_self, idx2)
    acc = _seg_sums(values, idx2)
    cnt = jnp.stack([cnt_r.reshape(NPAD), cnt_c.reshape(NPAD)])
    brow, bcol = _marg_mm(acc, cnt, theta_row, theta_col, theta_all, vsum)
    return _gather_finish(self_out, idx2, brow, bcol)


# speedup vs baseline: 1.0495x; 1.0495x over previous
import functools

import jax
import jax.numpy as jnp
from jax import lax
from jax.experimental import pallas as pl
from jax.experimental.pallas import tpu as pltpu
from jax.experimental.pallas import tpu_sc as plsc

N_R = 10000
NNZ = 320000
U = 128
EPS = 1e-10
NPAD = 10240
NHI = NPAD // 128
NBATCH = NNZ // 128

_mesh = plsc.VectorSubcoreMesh(core_axis_name="c", subcore_axis_name="s",
                               num_cores=2, num_subcores=16)


@functools.partial(
    pl.kernel,
    out_type=jax.ShapeDtypeStruct((2, NPAD, U), jnp.float32),
    mesh=_mesh,
    scratch_types=[
        pltpu.VMEM((2, 128, U), jnp.float32),
        pltpu.VMEM((2, 128), jnp.int32),
        pltpu.VMEM_SHARED((NPAD, U), jnp.float32),
        pltpu.SemaphoreType.DMA,
        pltpu.SemaphoreType.DMA,
    ],
)
def _seg_sums(values, idx2, acc_out, val_b, idx_b, acc_s, sl0, sl1):
    c = lax.axis_index("c")
    s = lax.axis_index("s")
    z16 = jnp.zeros((16,), jnp.float32)

    def _zero_val(r, _):
        for k in range(U // 16):
            val_b[0, r, pl.ds(k * 16, 16)] = z16
        return 0
    lax.fori_loop(0, 128, _zero_val, 0)

    for b in range(5):
        pltpu.sync_copy(val_b.at[0], acc_s.at[pl.ds(s * 640 + b * 128, 128), :])

    plsc.subcore_barrier()

    base = 156 * s + jnp.minimum(s, 4)
    sems = (sl0, sl1)

    def _b0(j):
        return jnp.minimum(base + j, NBATCH - 1) * 128

    def _issue(j, p):
        b0 = _b0(j)
        pltpu.async_copy(values.at[pl.ds(b0, 128), :], val_b.at[p], sems[p])
        pltpu.async_copy(idx2.at[c, pl.ds(b0, 128)], idx_b.at[p], sems[p])

    def _wait(j, p):
        b0 = _b0(j)
        pltpu.make_async_copy(values.at[pl.ds(b0, 128), :], val_b.at[p],
                              sems[p]).wait()
        pltpu.make_async_copy(idx2.at[c, pl.ds(b0, 128)], idx_b.at[p],
                              sems[p]).wait()

    def _scatter(p):
        pltpu.sync_copy(val_b.at[p], acc_s.at[idx_b.at[p]], add=True)

    _issue(0, 0)

    def _acc_body(t, _):
        j0 = 2 * t
        _issue(j0 + 1, 1)
        _wait(j0, 0)
        _scatter(0)
        _issue(j0 + 2, 0)
        _wait(j0 + 1, 1)
        _scatter(1)
        return 0
    lax.fori_loop(0, 78, _acc_body, 0)

    _wait(156, 0)

    @pl.when(s < 4)
    def _():
        _scatter(0)

    plsc.subcore_barrier()

    pltpu.sync_copy(acc_s.at[pl.ds(s * 640, 640), :],
                    acc_out.at[c, pl.ds(s * 640, 640), :])


def _self_mm_body(x_ref, w_ref, i2_ref, o_ref, s_ref, cr_ref, cc_ref):
    x = x_ref[...]
    o_ref[...] = jnp.dot(x, w_ref[...], preferred_element_type=jnp.float32)

    @pl.when(pl.program_id(0) == 0)
    def _():
        s_ref[...] = jnp.zeros_like(s_ref)
        cr_ref[...] = jnp.zeros_like(cr_ref)
        cc_ref[...] = jnp.zeros_like(cc_ref)

    s_ref[...] += jnp.sum(x, axis=0, keepdims=True)

    i2 = i2_ref[...]
    dn = (((0,), (0,)), ((), ()))
    for axis, ref in ((0, cr_ref), (1, cc_ref)):
        idx = i2[axis]
        oh_hi = (lax.broadcasted_iota(jnp.int32, (_MBLK, NHI), 1)
                 == (idx >> 7)[:, None]).astype(jnp.bfloat16)
        oh_lo = (lax.broadcasted_iota(jnp.int32, (_MBLK, U), 1)
                 == (idx & 127)[:, None]).astype(jnp.bfloat16)
        ref[...] += lax.dot_general(oh_hi, oh_lo, dn,
                                    preferred_element_type=jnp.float32)


_MBLK = 1280


def _self_mm(values, theta_self, idx2):
    return pl.pallas_call(
        _self_mm_body,
        grid=(NNZ // _MBLK,),
        in_specs=[
            pl.BlockSpec((_MBLK, U), lambda i: (i, 0)),
            pl.BlockSpec((U, U), lambda i: (0, 0)),
            pl.BlockSpec((2, _MBLK), lambda i: (0, i)),
        ],
        out_specs=[
            pl.BlockSpec((_MBLK, U), lambda i: (i, 0)),
            pl.BlockSpec((1, U), lambda i: (0, 0)),
            pl.BlockSpec((NHI, U), lambda i: (0, 0)),
            pl.BlockSpec((NHI, U), lambda i: (0, 0)),
        ],
        out_shape=[
            jax.ShapeDtypeStruct((NNZ, U), jnp.float32),
            jax.ShapeDtypeStruct((1, U), jnp.float32),
            jax.ShapeDtypeStruct((NHI, U), jnp.float32),
            jax.ShapeDtypeStruct((NHI, U), jnp.float32),
        ],
    )(values, theta_self, idx2)


def _marg_mm_body(a0, a1, cc, thr, thc, tha, vs, brow, bcol):
    bias = jnp.dot(vs[...], tha[...], preferred_element_type=jnp.float32)
    bias = bias * (0.5 / NNZ)
    cnt2 = cc[...]
    m0 = a0[0] / (cnt2[0][:, None] + EPS)
    m1 = a1[0] / (cnt2[1][:, None] + EPS)
    brow[...] = jnp.dot(m0, thr[...], preferred_element_type=jnp.float32) + bias
    bcol[...] = jnp.dot(m1, thc[...], preferred_element_type=jnp.float32) + bias


_BBLK = 1280


def _marg_mm(acc, cnt, theta_row, theta_col, theta_all, vsum):
    return pl.pallas_call(
        _marg_mm_body,
        grid=(NPAD // _BBLK,),
        in_specs=[
            pl.BlockSpec((1, _BBLK, U), lambda i: (0, i, 0)),
            pl.BlockSpec((1, _BBLK, U), lambda i: (1, i, 0)),
            pl.BlockSpec((2, _BBLK), lambda i: (0, i)),
            pl.BlockSpec((U, U), lambda i: (0, 0)),
            pl.BlockSpec((U, U), lambda i: (0, 0)),
            pl.BlockSpec((U, U), lambda i: (0, 0)),
            pl.BlockSpec((1, U), lambda i: (0, 0)),
        ],
        out_specs=[
            pl.BlockSpec((_BBLK, U), lambda i: (i, 0)),
            pl.BlockSpec((_BBLK, U), lambda i: (i, 0)),
        ],
        out_shape=[
            jax.ShapeDtypeStruct((NPAD, U), jnp.float32),
            jax.ShapeDtypeStruct((NPAD, U), jnp.float32),
        ],
    )(acc, acc, cnt, theta_row, theta_col, theta_all, vsum)


@functools.partial(
    pl.kernel,
    out_type=jax.ShapeDtypeStruct((NNZ, U), jnp.float32),
    mesh=_mesh,
    scratch_types=[
        pltpu.VMEM((2, 128, U), jnp.float32),
        pltpu.VMEM((2, 128, U), jnp.float32),
        pltpu.VMEM((2, 128, U), jnp.float32),
        pltpu.VMEM((81 * 128,), jnp.int32),
        pltpu.VMEM((81 * 128,), jnp.int32),
        pltpu.SemaphoreType.DMA,
        pltpu.SemaphoreType.DMA,
    ],
)
def _gather_finish(self_out, idx2, brow, bcol, out,
                   buf, g1, g2, idxr, idxc, sg0, sg1):
    c = lax.axis_index("c")
    s = lax.axis_index("s")
    w = s * 2 + c
    base = 78 * w + jnp.minimum(w, 4)
    start0 = jnp.minimum(base, NBATCH - 81) * 128
    zero16 = jnp.zeros((16,), jnp.float32)
    sems_g = (sg0, sg1)

    pltpu.sync_copy(idx2.at[0, pl.ds(start0, 81 * 128)], idxr)
    pltpu.sync_copy(idx2.at[1, pl.ds(start0, 81 * 128)], idxc)

    def _b0(j):
        return jnp.minimum(base + j, NBATCH - 1) * 128

    def _issue_loads(j, p):
        b0 = _b0(j)
        pltpu.async_copy(self_out.at[pl.ds(b0, 128), :], buf.at[p], sems_g[p])

    def _issue_gathers(j, p):
        off = _b0(j) - start0
        pltpu.async_copy(brow.at[idxr.at[pl.ds(off, 128)]], g1.at[p],
                         sems_g[p])
        pltpu.async_copy(bcol.at[idxc.at[pl.ds(off, 128)]], g2.at[p],
                         sems_g[p])

    def _compute_store(j, p):
        b0 = _b0(j)
        off = b0 - start0
        pltpu.make_async_copy(self_out.at[pl.ds(b0, 128), :], buf.at[p],
                              sems_g[p]).wait()
        pltpu.make_async_copy(brow.at[idxr.at[pl.ds(off, 128)]], g1.at[p],
                              sems_g[p]).wait()
        pltpu.make_async_copy(bcol.at[idxc.at[pl.ds(off, 128)]], g2.at[p],
                              sems_g[p]).wait()

        def _relu(r, _):
            for k in range(U // 16):
                sl = pl.ds(k * 16, 16)
                v = buf[p, r, sl] + g1[p, r, sl] + g2[p, r, sl]
                buf[p, r, sl] = jnp.maximum(v, zero16)
            return 0
        lax.fori_loop(0, 128, _relu, 0)
        pltpu.sync_copy(buf.at[p], out.at[pl.ds(b0, 128), :])

    _issue_loads(0, 0)
    _issue_gathers(0, 0)

    def _body(t, _):
        j0 = 2 * t
        _issue_loads(j0 + 1, 1)
        _issue_gathers(j0 + 1, 1)
        _compute_store(j0, 0)
        _issue_loads(j0 + 2, 0)
        _issue_gathers(j0 + 2, 0)
        _compute_store(j0 + 1, 1)
        return 0
    lax.fori_loop(0, 40, _body, 0)

    _compute_store(80, 0)


def kernel(values, indices, theta_self, theta_row, theta_col, theta_all):
    idx2 = jnp.transpose(indices.astype(jnp.int32))
    self_out, vsum, cnt_r, cnt_c = _self_mm(values, theta_self, idx2)
    acc = _seg_sums(values, idx2)
    cnt = jnp.stack([cnt_r.reshape(NPAD), cnt_c.reshape(NPAD)])
    brow, bcol = _marg_mm(acc, cnt, theta_row, theta_col, theta_all, vsum)
    return _gather_finish(self_out, idx2, brow, bcol)


# speedup vs baseline: 1.1903x; 1.1341x over previous
import functools

import jax
import jax.numpy as jnp
from jax import lax
from jax.experimental import pallas as pl
from jax.experimental.pallas import tpu as pltpu
from jax.experimental.pallas import tpu_sc as plsc

N_R = 10000
NNZ = 320000
U = 128
EPS = 1e-10
NPAD = 10240
NHI = NPAD // 128
NBATCH = NNZ // 128

_mesh = plsc.VectorSubcoreMesh(core_axis_name="c", subcore_axis_name="s",
                               num_cores=2, num_subcores=16)


@functools.partial(
    pl.kernel,
    out_type=jax.ShapeDtypeStruct((2, NPAD, U), jnp.float32),
    mesh=_mesh,
    scratch_types=[
        pltpu.VMEM((2, 128, U), jnp.float32),
        pltpu.VMEM((2, 128), jnp.int32),
        pltpu.VMEM_SHARED((NPAD, U), jnp.float32),
        pltpu.SemaphoreType.DMA,
        pltpu.SemaphoreType.DMA,
    ],
)
def _seg_sums(values, idx2, acc_out, val_b, idx_b, acc_s, sl0, sl1):
    c = lax.axis_index("c")
    s = lax.axis_index("s")
    z16 = jnp.zeros((16,), jnp.float32)

    def _zero_val(r, _):
        for k in range(U // 16):
            val_b[0, r, pl.ds(k * 16, 16)] = z16
        return 0
    lax.fori_loop(0, 128, _zero_val, 0)

    for b in range(5):
        pltpu.sync_copy(val_b.at[0], acc_s.at[pl.ds(s * 640 + b * 128, 128), :])

    plsc.subcore_barrier()

    base = 156 * s + jnp.minimum(s, 4)
    sems = (sl0, sl1)

    def _b0(j):
        return jnp.minimum(base + j, NBATCH - 1) * 128

    def _issue(j, p):
        b0 = _b0(j)
        pltpu.async_copy(values.at[pl.ds(b0, 128), :], val_b.at[p], sems[p])
        pltpu.async_copy(idx2.at[c, pl.ds(b0, 128)], idx_b.at[p], sems[p])

    def _wait(j, p):
        b0 = _b0(j)
        pltpu.make_async_copy(values.at[pl.ds(b0, 128), :], val_b.at[p],
                              sems[p]).wait()
        pltpu.make_async_copy(idx2.at[c, pl.ds(b0, 128)], idx_b.at[p],
                              sems[p]).wait()

    def _scatter(p):
        pltpu.sync_copy(val_b.at[p], acc_s.at[idx_b.at[p]], add=True)

    _issue(0, 0)

    def _acc_body(t, _):
        j0 = 2 * t
        _issue(j0 + 1, 1)
        _wait(j0, 0)
        _scatter(0)
        _issue(j0 + 2, 0)
        _wait(j0 + 1, 1)
        _scatter(1)
        return 0
    lax.fori_loop(0, 78, _acc_body, 0)

    _wait(156, 0)

    @pl.when(s < 4)
    def _():
        _scatter(0)

    plsc.subcore_barrier()

    pltpu.sync_copy(acc_s.at[pl.ds(s * 640, 640), :],
                    acc_out.at[c, pl.ds(s * 640, 640), :])


def _self_mm_body(x_ref, w_ref, i2_ref, o_ref, s_ref, cr_ref, cc_ref):
    x = x_ref[...]
    o_ref[...] = jnp.dot(x, w_ref[...], preferred_element_type=jnp.float32)

    @pl.when(pl.program_id(0) == 0)
    def _():
        s_ref[...] = jnp.zeros_like(s_ref)
        cr_ref[...] = jnp.zeros_like(cr_ref)
        cc_ref[...] = jnp.zeros_like(cc_ref)

    s_ref[...] += jnp.sum(x, axis=0, keepdims=True)

    i2 = i2_ref[...]
    dn = (((0,), (0,)), ((), ()))
    for axis, ref in ((0, cr_ref), (1, cc_ref)):
        idx = i2[axis]
        oh_hi = (lax.broadcasted_iota(jnp.int32, (_MBLK, NHI), 1)
                 == (idx >> 7)[:, None]).astype(jnp.bfloat16)
        oh_lo = (lax.broadcasted_iota(jnp.int32, (_MBLK, U), 1)
                 == (idx & 127)[:, None]).astype(jnp.bfloat16)
        ref[...] += lax.dot_general(oh_hi, oh_lo, dn,
                                    preferred_element_type=jnp.float32)


_MBLK = 3200


def _self_mm(values, theta_self, idx2):
    return pl.pallas_call(
        _self_mm_body,
        grid=(NNZ // _MBLK,),
        in_specs=[
            pl.BlockSpec((_MBLK, U), lambda i: (i, 0)),
            pl.BlockSpec((U, U), lambda i: (0, 0)),
            pl.BlockSpec((2, _MBLK), lambda i: (0, i)),
        ],
        out_specs=[
            pl.BlockSpec((_MBLK, U), lambda i: (i, 0)),
            pl.BlockSpec((1, U), lambda i: (0, 0)),
            pl.BlockSpec((NHI, U), lambda i: (0, 0)),
            pl.BlockSpec((NHI, U), lambda i: (0, 0)),
        ],
        out_shape=[
            jax.ShapeDtypeStruct((NNZ, U), jnp.float32),
            jax.ShapeDtypeStruct((1, U), jnp.float32),
            jax.ShapeDtypeStruct((NHI, U), jnp.float32),
            jax.ShapeDtypeStruct((NHI, U), jnp.float32),
        ],
    )(values, theta_self, idx2)


def _marg_mm_body(a0, a1, cc, thr, thc, tha, vs, brow, bcol):
    bias = jnp.dot(vs[...], tha[...], preferred_element_type=jnp.float32)
    bias = bias * (0.5 / NNZ)
    cnt2 = cc[...]
    m0 = a0[0] / (cnt2[0][:, None] + EPS)
    m1 = a1[0] / (cnt2[1][:, None] + EPS)
    brow[...] = jnp.dot(m0, thr[...], preferred_element_type=jnp.float32) + bias
    bcol[...] = jnp.dot(m1, thc[...], preferred_element_type=jnp.float32) + bias


_BBLK = 1280


def _marg_mm(acc, cnt, theta_row, theta_col, theta_all, vsum):
    return pl.pallas_call(
        _marg_mm_body,
        grid=(NPAD // _BBLK,),
        in_specs=[
            pl.BlockSpec((1, _BBLK, U), lambda i: (0, i, 0)),
            pl.BlockSpec((1, _BBLK, U), lambda i: (1, i, 0)),
            pl.BlockSpec((2, _BBLK), lambda i: (0, i)),
            pl.BlockSpec((U, U), lambda i: (0, 0)),
            pl.BlockSpec((U, U), lambda i: (0, 0)),
            pl.BlockSpec((U, U), lambda i: (0, 0)),
            pl.BlockSpec((1, U), lambda i: (0, 0)),
        ],
        out_specs=[
            pl.BlockSpec((_BBLK, U), lambda i: (i, 0)),
            pl.BlockSpec((_BBLK, U), lambda i: (i, 0)),
        ],
        out_shape=[
            jax.ShapeDtypeStruct((NPAD, U), jnp.float32),
            jax.ShapeDtypeStruct((NPAD, U), jnp.float32),
        ],
    )(acc, acc, cnt, theta_row, theta_col, theta_all, vsum)


@functools.partial(
    pl.kernel,
    out_type=jax.ShapeDtypeStruct((NNZ, U), jnp.float32),
    mesh=_mesh,
    scratch_types=[
        pltpu.VMEM((2, 128, U), jnp.float32),
        pltpu.VMEM((2, 128, U), jnp.float32),
        pltpu.VMEM((2, 128, U), jnp.float32),
        pltpu.VMEM((81 * 128,), jnp.int32),
        pltpu.VMEM((81 * 128,), jnp.int32),
        pltpu.SemaphoreType.DMA,
        pltpu.SemaphoreType.DMA,
    ],
)
def _gather_finish(self_out, idx2, brow, bcol, out,
                   buf, g1, g2, idxr, idxc, sg0, sg1):
    c = lax.axis_index("c")
    s = lax.axis_index("s")
    w = s * 2 + c
    base = 78 * w + jnp.minimum(w, 4)
    start0 = jnp.minimum(base, NBATCH - 81) * 128
    zero16 = jnp.zeros((16,), jnp.float32)
    sems_g = (sg0, sg1)

    pltpu.sync_copy(idx2.at[0, pl.ds(start0, 81 * 128)], idxr)
    pltpu.sync_copy(idx2.at[1, pl.ds(start0, 81 * 128)], idxc)

    def _b0(j):
        return jnp.minimum(base + j, NBATCH - 1) * 128

    def _issue_loads(j, p):
        b0 = _b0(j)
        pltpu.async_copy(self_out.at[pl.ds(b0, 128), :], buf.at[p], sems_g[p])

    def _issue_gathers(j, p):
        off = _b0(j) - start0
        pltpu.async_copy(brow.at[idxr.at[pl.ds(off, 128)]], g1.at[p],
                         sems_g[p])
        pltpu.async_copy(bcol.at[idxc.at[pl.ds(off, 128)]], g2.at[p],
                         sems_g[p])

    def _compute_store(j, p):
        b0 = _b0(j)
        off = b0 - start0
        pltpu.make_async_copy(self_out.at[pl.ds(b0, 128), :], buf.at[p],
                              sems_g[p]).wait()
        pltpu.make_async_copy(brow.at[idxr.at[pl.ds(off, 128)]], g1.at[p],
                              sems_g[p]).wait()
        pltpu.make_async_copy(bcol.at[idxc.at[pl.ds(off, 128)]], g2.at[p],
                              sems_g[p]).wait()

        def _relu(r, _):
            for k in range(U // 16):
                sl = pl.ds(k * 16, 16)
                v = buf[p, r, sl] + g1[p, r, sl] + g2[p, r, sl]
                buf[p, r, sl] = jnp.maximum(v, zero16)
            return 0
        lax.fori_loop(0, 128, _relu, 0)
        pltpu.sync_copy(buf.at[p], out.at[pl.ds(b0, 128), :])

    _issue_loads(0, 0)
    _issue_gathers(0, 0)

    def _body(t, _):
        j0 = 2 * t
        _issue_loads(j0 + 1, 1)
        _issue_gathers(j0 + 1, 1)
        _compute_store(j0, 0)
        _issue_loads(j0 + 2, 0)
        _issue_gathers(j0 + 2, 0)
        _compute_store(j0 + 1, 1)
        return 0
    lax.fori_loop(0, 40, _body, 0)

    _compute_store(80, 0)


def kernel(values, indices, theta_self, theta_row, theta_col, theta_all):
    idx2 = jnp.transpose(indices.astype(jnp.int32))
    self_out, vsum, cnt_r, cnt_c = _self_mm(values, theta_self, idx2)
    acc = _seg_sums(values, idx2)
    cnt = jnp.stack([cnt_r.reshape(NPAD), cnt_c.reshape(NPAD)])
    brow, bcol = _marg_mm(acc, cnt, theta_row, theta_col, theta_all, vsum)
    return _gather_finish(self_out, idx2, brow, bcol)


# speedup vs baseline: 1.2360x; 1.0384x over previous
import functools

import jax
import jax.numpy as jnp
from jax import lax
from jax.experimental import pallas as pl
from jax.experimental.pallas import tpu as pltpu
from jax.experimental.pallas import tpu_sc as plsc

N_R = 10000
NNZ = 320000
U = 128
EPS = 1e-10
NPAD = 10240
NHI = NPAD // 128
NBATCH = NNZ // 128

_mesh = plsc.VectorSubcoreMesh(core_axis_name="c", subcore_axis_name="s",
                               num_cores=2, num_subcores=16)


@functools.partial(
    pl.kernel,
    out_type=jax.ShapeDtypeStruct((2, NPAD, U), jnp.float32),
    mesh=_mesh,
    scratch_types=[
        pltpu.VMEM((2, 128, U), jnp.float32),
        pltpu.VMEM((2, 128), jnp.int32),
        pltpu.VMEM_SHARED((NPAD, U), jnp.float32),
        pltpu.SemaphoreType.DMA,
        pltpu.SemaphoreType.DMA,
    ],
)
def _seg_sums(values, idx2, acc_out, val_b, idx_b, acc_s, sl0, sl1):
    c = lax.axis_index("c")
    s = lax.axis_index("s")
    z16 = jnp.zeros((16,), jnp.float32)

    def _zero_val(r, _):
        for k in range(U // 16):
            val_b[0, r, pl.ds(k * 16, 16)] = z16
        return 0
    lax.fori_loop(0, 128, _zero_val, 0)

    for b in range(5):
        pltpu.sync_copy(val_b.at[0], acc_s.at[pl.ds(s * 640 + b * 128, 128), :])

    plsc.subcore_barrier()

    base = 156 * s + jnp.minimum(s, 4)
    sems = (sl0, sl1)

    def _b0(j):
        return jnp.minimum(base + j, NBATCH - 1) * 128

    def _issue(j, p):
        b0 = _b0(j)
        pltpu.async_copy(values.at[pl.ds(b0, 128), :], val_b.at[p], sems[p])
        pltpu.async_copy(idx2.at[c, pl.ds(b0, 128)], idx_b.at[p], sems[p])

    def _wait(j, p):
        b0 = _b0(j)
        pltpu.make_async_copy(values.at[pl.ds(b0, 128), :], val_b.at[p],
                              sems[p]).wait()
        pltpu.make_async_copy(idx2.at[c, pl.ds(b0, 128)], idx_b.at[p],
                              sems[p]).wait()

    def _scatter(p):
        pltpu.sync_copy(val_b.at[p], acc_s.at[idx_b.at[p]], add=True)

    _issue(0, 0)

    def _acc_body(t, _):
        j0 = 2 * t
        _issue(j0 + 1, 1)
        _wait(j0, 0)
        _scatter(0)
        _issue(j0 + 2, 0)
        _wait(j0 + 1, 1)
        _scatter(1)
        return 0
    lax.fori_loop(0, 78, _acc_body, 0)

    _wait(156, 0)

    @pl.when(s < 4)
    def _():
        _scatter(0)

    plsc.subcore_barrier()

    pltpu.sync_copy(acc_s.at[pl.ds(s * 640, 640), :],
                    acc_out.at[c, pl.ds(s * 640, 640), :])


def _self_mm_body(x_ref, w_ref, i2_ref, o_ref, s_ref, cr_ref, cc_ref):
    x = x_ref[...]
    o_ref[...] = jnp.dot(x, w_ref[...], preferred_element_type=jnp.float32)

    @pl.when(pl.program_id(0) == 0)
    def _():
        s_ref[...] = jnp.zeros_like(s_ref)
        cr_ref[...] = jnp.zeros_like(cr_ref)
        cc_ref[...] = jnp.zeros_like(cc_ref)

    s_ref[...] += jnp.sum(x, axis=0, keepdims=True)

    i2 = i2_ref[...]
    dn = (((0,), (0,)), ((), ()))
    for axis, ref in ((0, cr_ref), (1, cc_ref)):
        idx = i2[axis]
        oh_hi = (lax.broadcasted_iota(jnp.int32, (_MBLK, NHI), 1)
                 == (idx >> 7)[:, None]).astype(jnp.bfloat16)
        oh_lo = (lax.broadcasted_iota(jnp.int32, (_MBLK, U), 1)
                 == (idx & 127)[:, None]).astype(jnp.bfloat16)
        ref[...] += lax.dot_general(oh_hi, oh_lo, dn,
                                    preferred_element_type=jnp.float32)


_MBLK = 6400


def _self_mm(values, theta_self, idx2):
    return pl.pallas_call(
        _self_mm_body,
        grid=(NNZ // _MBLK,),
        in_specs=[
            pl.BlockSpec((_MBLK, U), lambda i: (i, 0)),
            pl.BlockSpec((U, U), lambda i: (0, 0)),
            pl.BlockSpec((2, _MBLK), lambda i: (0, i)),
        ],
        out_specs=[
            pl.BlockSpec((_MBLK, U), lambda i: (i, 0)),
            pl.BlockSpec((1, U), lambda i: (0, 0)),
            pl.BlockSpec((NHI, U), lambda i: (0, 0)),
            pl.BlockSpec((NHI, U), lambda i: (0, 0)),
        ],
        out_shape=[
            jax.ShapeDtypeStruct((NNZ, U), jnp.float32),
            jax.ShapeDtypeStruct((1, U), jnp.float32),
            jax.ShapeDtypeStruct((NHI, U), jnp.float32),
            jax.ShapeDtypeStruct((NHI, U), jnp.float32),
        ],
    )(values, theta_self, idx2)


def _marg_mm_body(a0, a1, cc, thr, thc, tha, vs, brow, bcol):
    bias = jnp.dot(vs[...], tha[...], preferred_element_type=jnp.float32)
    bias = bias * (0.5 / NNZ)
    cnt2 = cc[...]
    m0 = a0[0] / (cnt2[0][:, None] + EPS)
    m1 = a1[0] / (cnt2[1][:, None] + EPS)
    brow[...] = jnp.dot(m0, thr[...], preferred_element_type=jnp.float32) + bias
    bcol[...] = jnp.dot(m1, thc[...], preferred_element_type=jnp.float32) + bias


_BBLK = 2048


def _marg_mm(acc, cnt, theta_row, theta_col, theta_all, vsum):
    return pl.pallas_call(
        _marg_mm_body,
        grid=(NPAD // _BBLK,),
        in_specs=[
            pl.BlockSpec((1, _BBLK, U), lambda i: (0, i, 0)),
            pl.BlockSpec((1, _BBLK, U), lambda i: (1, i, 0)),
            pl.BlockSpec((2, _BBLK), lambda i: (0, i)),
            pl.BlockSpec((U, U), lambda i: (0, 0)),
            pl.BlockSpec((U, U), lambda i: (0, 0)),
            pl.BlockSpec((U, U), lambda i: (0, 0)),
            pl.BlockSpec((1, U), lambda i: (0, 0)),
        ],
        out_specs=[
            pl.BlockSpec((_BBLK, U), lambda i: (i, 0)),
            pl.BlockSpec((_BBLK, U), lambda i: (i, 0)),
        ],
        out_shape=[
            jax.ShapeDtypeStruct((NPAD, U), jnp.float32),
            jax.ShapeDtypeStruct((NPAD, U), jnp.float32),
        ],
    )(acc, acc, cnt, theta_row, theta_col, theta_all, vsum)


@functools.partial(
    pl.kernel,
    out_type=jax.ShapeDtypeStruct((NNZ, U), jnp.float32),
    mesh=_mesh,
    scratch_types=[
        pltpu.VMEM((2, 128, U), jnp.float32),
        pltpu.VMEM((2, 128, U), jnp.float32),
        pltpu.VMEM((2, 128, U), jnp.float32),
        pltpu.VMEM((81 * 128,), jnp.int32),
        pltpu.VMEM((81 * 128,), jnp.int32),
        pltpu.SemaphoreType.DMA,
        pltpu.SemaphoreType.DMA,
    ],
)
def _gather_finish(self_out, idx2, brow, bcol, out,
                   buf, g1, g2, idxr, idxc, sg0, sg1):
    c = lax.axis_index("c")
    s = lax.axis_index("s")
    w = s * 2 + c
    base = 78 * w + jnp.minimum(w, 4)
    start0 = jnp.minimum(base, NBATCH - 81) * 128
    zero16 = jnp.zeros((16,), jnp.float32)
    sems_g = (sg0, sg1)

    pltpu.sync_copy(idx2.at[0, pl.ds(start0, 81 * 128)], idxr)
    pltpu.sync_copy(idx2.at[1, pl.ds(start0, 81 * 128)], idxc)

    def _b0(j):
        return jnp.minimum(base + j, NBATCH - 1) * 128

    def _issue_loads(j, p):
        b0 = _b0(j)
        pltpu.async_copy(self_out.at[pl.ds(b0, 128), :], buf.at[p], sems_g[p])

    def _issue_gathers(j, p):
        off = _b0(j) - start0
        pltpu.async_copy(brow.at[idxr.at[pl.ds(off, 128)]], g1.at[p],
                         sems_g[p])
        pltpu.async_copy(bcol.at[idxc.at[pl.ds(off, 128)]], g2.at[p],
                         sems_g[p])

    def _compute_store(j, p):
        b0 = _b0(j)
        off = b0 - start0
        pltpu.make_async_copy(self_out.at[pl.ds(b0, 128), :], buf.at[p],
                              sems_g[p]).wait()
        pltpu.make_async_copy(brow.at[idxr.at[pl.ds(off, 128)]], g1.at[p],
                              sems_g[p]).wait()
        pltpu.make_async_copy(bcol.at[idxc.at[pl.ds(off, 128)]], g2.at[p],
                              sems_g[p]).wait()

        def _relu(r, _):
            for k in range(U // 16):
                sl = pl.ds(k * 16, 16)
                v = buf[p, r, sl] + g1[p, r, sl] + g2[p, r, sl]
                buf[p, r, sl] = jnp.maximum(v, zero16)
            return 0
        lax.fori_loop(0, 128, _relu, 0)
        pltpu.sync_copy(buf.at[p], out.at[pl.ds(b0, 128), :])

    _issue_loads(0, 0)
    _issue_gathers(0, 0)

    def _body(t, _):
        j0 = 2 * t
        _issue_loads(j0 + 1, 1)
        _issue_gathers(j0 + 1, 1)
        _compute_store(j0, 0)
        _issue_loads(j0 + 2, 0)
        _issue_gathers(j0 + 2, 0)
        _compute_store(j0 + 1, 1)
        return 0
    lax.fori_loop(0, 40, _body, 0)

    _compute_store(80, 0)


def kernel(values, indices, theta_self, theta_row, theta_col, theta_all):
    idx2 = jnp.transpose(indices.astype(jnp.int32))
    self_out, vsum, cnt_r, cnt_c = _self_mm(values, theta_self, idx2)
    acc = _seg_sums(values, idx2)
    cnt = jnp.stack([cnt_r.reshape(NPAD), cnt_c.reshape(NPAD)])
    brow, bcol = _marg_mm(acc, cnt, theta_row, theta_col, theta_all, vsum)
    return _gather_finish(self_out, idx2, brow, bcol)


# speedup vs baseline: 1.2643x; 1.0229x over previous
import functools

import jax
import jax.numpy as jnp
from jax import lax
from jax.experimental import pallas as pl
from jax.experimental.pallas import tpu as pltpu
from jax.experimental.pallas import tpu_sc as plsc

N_R = 10000
NNZ = 320000
U = 128
EPS = 1e-10
NPAD = 10240
NHI = NPAD // 128
NBATCH = NNZ // 128

_mesh = plsc.VectorSubcoreMesh(core_axis_name="c", subcore_axis_name="s",
                               num_cores=2, num_subcores=16)


@functools.partial(
    pl.kernel,
    out_type=jax.ShapeDtypeStruct((2, NPAD, U), jnp.float32),
    mesh=_mesh,
    scratch_types=[
        pltpu.VMEM((2, 128, U), jnp.float32),
        pltpu.VMEM((2, 128), jnp.int32),
        pltpu.VMEM_SHARED((NPAD, U), jnp.float32),
        pltpu.SemaphoreType.DMA,
        pltpu.SemaphoreType.DMA,
    ],
)
def _seg_sums(values, idx2, acc_out, val_b, idx_b, acc_s, sl0, sl1):
    c = lax.axis_index("c")
    s = lax.axis_index("s")
    z16 = jnp.zeros((16,), jnp.float32)

    def _zero_val(r, _):
        for k in range(U // 16):
            val_b[0, r, pl.ds(k * 16, 16)] = z16
        return 0
    lax.fori_loop(0, 128, _zero_val, 0)

    for b in range(5):
        pltpu.sync_copy(val_b.at[0], acc_s.at[pl.ds(s * 640 + b * 128, 128), :])

    plsc.subcore_barrier()

    base = 156 * s + jnp.minimum(s, 4)
    sems = (sl0, sl1)

    def _b0(j):
        return jnp.minimum(base + j, NBATCH - 1) * 128

    def _issue(j, p):
        b0 = _b0(j)
        pltpu.async_copy(values.at[pl.ds(b0, 128), :], val_b.at[p], sems[p])
        pltpu.async_copy(idx2.at[c, pl.ds(b0, 128)], idx_b.at[p], sems[p])

    def _wait(j, p):
        b0 = _b0(j)
        pltpu.make_async_copy(values.at[pl.ds(b0, 128), :], val_b.at[p],
                              sems[p]).wait()
        pltpu.make_async_copy(idx2.at[c, pl.ds(b0, 128)], idx_b.at[p],
                              sems[p]).wait()

    def _scatter(p):
        pltpu.sync_copy(val_b.at[p], acc_s.at[idx_b.at[p]], add=True)

    _issue(0, 0)

    def _acc_body(t, _):
        j0 = 2 * t
        _issue(j0 + 1, 1)
        _wait(j0, 0)
        _scatter(0)
        _issue(j0 + 2, 0)
        _wait(j0 + 1, 1)
        _scatter(1)
        return 0
    lax.fori_loop(0, 78, _acc_body, 0)

    _wait(156, 0)

    @pl.when(s < 4)
    def _():
        _scatter(0)

    plsc.subcore_barrier()

    pltpu.sync_copy(acc_s.at[pl.ds(s * 640, 640), :],
                    acc_out.at[c, pl.ds(s * 640, 640), :])


def _self_mm_body(x_ref, w_ref, i2_ref, o_ref, s_ref, cr_ref, cc_ref):
    x = x_ref[...]
    o_ref[...] = jnp.dot(x, w_ref[...], preferred_element_type=jnp.float32)

    @pl.when(pl.program_id(0) == 0)
    def _():
        s_ref[...] = jnp.zeros_like(s_ref)
        cr_ref[...] = jnp.zeros_like(cr_ref)
        cc_ref[...] = jnp.zeros_like(cc_ref)

    s_ref[...] += jnp.sum(x, axis=0, keepdims=True)

    i2 = i2_ref[...]
    dn = (((0,), (0,)), ((), ()))
    for axis, ref in ((0, cr_ref), (1, cc_ref)):
        idx = i2[axis]
        oh_hi = (lax.broadcasted_iota(jnp.int32, (_MBLK, NHI), 1)
                 == (idx >> 7)[:, None]).astype(jnp.bfloat16)
        oh_lo = (lax.broadcasted_iota(jnp.int32, (_MBLK, U), 1)
                 == (idx & 127)[:, None]).astype(jnp.bfloat16)
        ref[...] += lax.dot_general(oh_hi, oh_lo, dn,
                                    preferred_element_type=jnp.float32)


_MBLK = 12800


def _self_mm(values, theta_self, idx2):
    return pl.pallas_call(
        _self_mm_body,
        grid=(NNZ // _MBLK,),
        in_specs=[
            pl.BlockSpec((_MBLK, U), lambda i: (i, 0)),
            pl.BlockSpec((U, U), lambda i: (0, 0)),
            pl.BlockSpec((2, _MBLK), lambda i: (0, i)),
        ],
        out_specs=[
            pl.BlockSpec((_MBLK, U), lambda i: (i, 0)),
            pl.BlockSpec((1, U), lambda i: (0, 0)),
            pl.BlockSpec((NHI, U), lambda i: (0, 0)),
            pl.BlockSpec((NHI, U), lambda i: (0, 0)),
        ],
        out_shape=[
            jax.ShapeDtypeStruct((NNZ, U), jnp.float32),
            jax.ShapeDtypeStruct((1, U), jnp.float32),
            jax.ShapeDtypeStruct((NHI, U), jnp.float32),
            jax.ShapeDtypeStruct((NHI, U), jnp.float32),
        ],
    )(values, theta_self, idx2)


def _marg_mm_body(a0, a1, cc, thr, thc, tha, vs, brow, bcol):
    bias = jnp.dot(vs[...], tha[...], preferred_element_type=jnp.float32)
    bias = bias * (0.5 / NNZ)
    cnt2 = cc[...]
    m0 = a0[0] / (cnt2[0][:, None] + EPS)
    m1 = a1[0] / (cnt2[1][:, None] + EPS)
    brow[...] = jnp.dot(m0, thr[...], preferred_element_type=jnp.float32) + bias
    bcol[...] = jnp.dot(m1, thc[...], preferred_element_type=jnp.float32) + bias


_BBLK = 5120


def _marg_mm(acc, cnt, theta_row, theta_col, theta_all, vsum):
    return pl.pallas_call(
        _marg_mm_body,
        grid=(NPAD // _BBLK,),
        in_specs=[
            pl.BlockSpec((1, _BBLK, U), lambda i: (0, i, 0)),
            pl.BlockSpec((1, _BBLK, U), lambda i: (1, i, 0)),
            pl.BlockSpec((2, _BBLK), lambda i: (0, i)),
            pl.BlockSpec((U, U), lambda i: (0, 0)),
            pl.BlockSpec((U, U), lambda i: (0, 0)),
            pl.BlockSpec((U, U), lambda i: (0, 0)),
            pl.BlockSpec((1, U), lambda i: (0, 0)),
        ],
        out_specs=[
            pl.BlockSpec((_BBLK, U), lambda i: (i, 0)),
            pl.BlockSpec((_BBLK, U), lambda i: (i, 0)),
        ],
        out_shape=[
            jax.ShapeDtypeStruct((NPAD, U), jnp.float32),
            jax.ShapeDtypeStruct((NPAD, U), jnp.float32),
        ],
    )(acc, acc, cnt, theta_row, theta_col, theta_all, vsum)


@functools.partial(
    pl.kernel,
    out_type=jax.ShapeDtypeStruct((NNZ, U), jnp.float32),
    mesh=_mesh,
    scratch_types=[
        pltpu.VMEM((2, 128, U), jnp.float32),
        pltpu.VMEM((2, 128, U), jnp.float32),
        pltpu.VMEM((2, 128, U), jnp.float32),
        pltpu.VMEM((81 * 128,), jnp.int32),
        pltpu.VMEM((81 * 128,), jnp.int32),
        pltpu.SemaphoreType.DMA,
        pltpu.SemaphoreType.DMA,
    ],
)
def _gather_finish(self_out, idx2, brow, bcol, out,
                   buf, g1, g2, idxr, idxc, sg0, sg1):
    c = lax.axis_index("c")
    s = lax.axis_index("s")
    w = s * 2 + c
    base = 78 * w + jnp.minimum(w, 4)
    start0 = jnp.minimum(base, NBATCH - 81) * 128
    zero16 = jnp.zeros((16,), jnp.float32)
    sems_g = (sg0, sg1)

    pltpu.sync_copy(idx2.at[0, pl.ds(start0, 81 * 128)], idxr)
    pltpu.sync_copy(idx2.at[1, pl.ds(start0, 81 * 128)], idxc)

    def _b0(j):
        return jnp.minimum(base + j, NBATCH - 1) * 128

    def _issue_loads(j, p):
        b0 = _b0(j)
        pltpu.async_copy(self_out.at[pl.ds(b0, 128), :], buf.at[p], sems_g[p])

    def _issue_gathers(j, p):
        off = _b0(j) - start0
        pltpu.async_copy(brow.at[idxr.at[pl.ds(off, 128)]], g1.at[p],
                         sems_g[p])
        pltpu.async_copy(bcol.at[idxc.at[pl.ds(off, 128)]], g2.at[p],
                         sems_g[p])

    def _compute_store(j, p):
        b0 = _b0(j)
        off = b0 - start0
        pltpu.make_async_copy(self_out.at[pl.ds(b0, 128), :], buf.at[p],
                              sems_g[p]).wait()
        pltpu.make_async_copy(brow.at[idxr.at[pl.ds(off, 128)]], g1.at[p],
                              sems_g[p]).wait()
        pltpu.make_async_copy(bcol.at[idxc.at[pl.ds(off, 128)]], g2.at[p],
                              sems_g[p]).wait()

        def _relu(r, _):
            for k in range(U // 16):
                sl = pl.ds(k * 16, 16)
                v = buf[p, r, sl] + g1[p, r, sl] + g2[p, r, sl]
                buf[p, r, sl] = jnp.maximum(v, zero16)
            return 0
        lax.fori_loop(0, 128, _relu, 0)
        pltpu.sync_copy(buf.at[p], out.at[pl.ds(b0, 128), :])

    _issue_loads(0, 0)
    _issue_gathers(0, 0)

    def _body(t, _):
        j0 = 2 * t
        _issue_loads(j0 + 1, 1)
        _issue_gathers(j0 + 1, 1)
        _compute_store(j0, 0)
        _issue_loads(j0 + 2, 0)
        _issue_gathers(j0 + 2, 0)
        _compute_store(j0 + 1, 1)
        return 0
    lax.fori_loop(0, 40, _body, 0)

    _compute_store(80, 0)


def kernel(values, indices, theta_self, theta_row, theta_col, theta_all):
    idx2 = jnp.transpose(indices.astype(jnp.int32))
    self_out, vsum, cnt_r, cnt_c = _self_mm(values, theta_self, idx2)
    acc = _seg_sums(values, idx2)
    cnt = jnp.stack([cnt_r.reshape(NPAD), cnt_c.reshape(NPAD)])
    brow, bcol = _marg_mm(acc, cnt, theta_row, theta_col, theta_all, vsum)
    return _gather_finish(self_out, idx2, brow, bcol)


# speedup vs baseline: 1.2677x; 1.0027x over previous
import functools

import jax
import jax.numpy as jnp
from jax import lax
from jax.experimental import pallas as pl
from jax.experimental.pallas import tpu as pltpu
from jax.experimental.pallas import tpu_sc as plsc

N_R = 10000
NNZ = 320000
U = 128
EPS = 1e-10
NPAD = 10240
NHI = NPAD // 128
NBATCH = NNZ // 128

_mesh = plsc.VectorSubcoreMesh(core_axis_name="c", subcore_axis_name="s",
                               num_cores=2, num_subcores=16)


@functools.partial(
    pl.kernel,
    out_type=jax.ShapeDtypeStruct((2, NPAD, U), jnp.float32),
    mesh=_mesh,
    scratch_types=[
        pltpu.VMEM((2, 128, U), jnp.float32),
        pltpu.VMEM((2, 128), jnp.int32),
        pltpu.VMEM_SHARED((NPAD, U), jnp.float32),
        pltpu.SemaphoreType.DMA,
        pltpu.SemaphoreType.DMA,
    ],
)
def _seg_sums(values, idx2, acc_out, val_b, idx_b, acc_s, sl0, sl1):
    c = lax.axis_index("c")
    s = lax.axis_index("s")
    z16 = jnp.zeros((16,), jnp.float32)

    def _zero_val(r, _):
        for k in range(U // 16):
            val_b[0, r, pl.ds(k * 16, 16)] = z16
        return 0
    lax.fori_loop(0, 128, _zero_val, 0)

    for b in range(5):
        pltpu.sync_copy(val_b.at[0], acc_s.at[pl.ds(s * 640 + b * 128, 128), :])

    plsc.subcore_barrier()

    base = 156 * s + jnp.minimum(s, 4)
    sems = (sl0, sl1)

    def _b0(j):
        return jnp.minimum(base + j, NBATCH - 1) * 128

    def _issue(j, p):
        b0 = _b0(j)
        pltpu.async_copy(values.at[pl.ds(b0, 128), :], val_b.at[p], sems[p])
        pltpu.async_copy(idx2.at[c, pl.ds(b0, 128)], idx_b.at[p], sems[p])

    def _wait(j, p):
        b0 = _b0(j)
        pltpu.make_async_copy(values.at[pl.ds(b0, 128), :], val_b.at[p],
                              sems[p]).wait()
        pltpu.make_async_copy(idx2.at[c, pl.ds(b0, 128)], idx_b.at[p],
                              sems[p]).wait()

    def _scatter(p):
        pltpu.sync_copy(val_b.at[p], acc_s.at[idx_b.at[p]], add=True)

    _issue(0, 0)

    def _acc_body(t, _):
        j0 = 2 * t
        _issue(j0 + 1, 1)
        _wait(j0, 0)
        _scatter(0)
        _issue(j0 + 2, 0)
        _wait(j0 + 1, 1)
        _scatter(1)
        return 0
    lax.fori_loop(0, 78, _acc_body, 0)

    _wait(156, 0)

    @pl.when(s < 4)
    def _():
        _scatter(0)

    plsc.subcore_barrier()

    pltpu.sync_copy(acc_s.at[pl.ds(s * 640, 640), :],
                    acc_out.at[c, pl.ds(s * 640, 640), :])


def _self_mm_body(x_ref, w_ref, i2_ref, o_ref, s_ref, cr_ref, cc_ref):
    x = x_ref[...]
    o_ref[...] = jnp.dot(x, w_ref[...], preferred_element_type=jnp.float32)

    @pl.when(pl.program_id(0) == 0)
    def _():
        s_ref[...] = jnp.zeros_like(s_ref)
        cr_ref[...] = jnp.zeros_like(cr_ref)
        cc_ref[...] = jnp.zeros_like(cc_ref)

    s_ref[...] += jnp.sum(x, axis=0, keepdims=True)

    i2 = i2_ref[...]
    dn = (((0,), (0,)), ((), ()))
    for axis, ref in ((0, cr_ref), (1, cc_ref)):
        idx = i2[axis]
        oh_hi = (lax.broadcasted_iota(jnp.int32, (_MBLK, NHI), 1)
                 == (idx >> 7)[:, None]).astype(jnp.bfloat16)
        oh_lo = (lax.broadcasted_iota(jnp.int32, (_MBLK, U), 1)
                 == (idx & 127)[:, None]).astype(jnp.bfloat16)
        ref[...] += lax.dot_general(oh_hi, oh_lo, dn,
                                    preferred_element_type=jnp.float32)


_MBLK = 16000


def _self_mm(values, theta_self, idx2):
    return pl.pallas_call(
        _self_mm_body,
        grid=(NNZ // _MBLK,),
        in_specs=[
            pl.BlockSpec((_MBLK, U), lambda i: (i, 0)),
            pl.BlockSpec((U, U), lambda i: (0, 0)),
            pl.BlockSpec((2, _MBLK), lambda i: (0, i)),
        ],
        out_specs=[
            pl.BlockSpec((_MBLK, U), lambda i: (i, 0)),
            pl.BlockSpec((1, U), lambda i: (0, 0)),
            pl.BlockSpec((NHI, U), lambda i: (0, 0)),
            pl.BlockSpec((NHI, U), lambda i: (0, 0)),
        ],
        out_shape=[
            jax.ShapeDtypeStruct((NNZ, U), jnp.float32),
            jax.ShapeDtypeStruct((1, U), jnp.float32),
            jax.ShapeDtypeStruct((NHI, U), jnp.float32),
            jax.ShapeDtypeStruct((NHI, U), jnp.float32),
        ],
    )(values, theta_self, idx2)


def _marg_mm_body(a0, a1, cc, thr, thc, tha, vs, brow, bcol):
    bias = jnp.dot(vs[...], tha[...], preferred_element_type=jnp.float32)
    bias = bias * (0.5 / NNZ)
    cnt2 = cc[...]
    m0 = a0[0] / (cnt2[0][:, None] + EPS)
    m1 = a1[0] / (cnt2[1][:, None] + EPS)
    brow[...] = jnp.dot(m0, thr[...], preferred_element_type=jnp.float32) + bias
    bcol[...] = jnp.dot(m1, thc[...], preferred_element_type=jnp.float32) + bias


_BBLK = 10240


def _marg_mm(acc, cnt, theta_row, theta_col, theta_all, vsum):
    return pl.pallas_call(
        _marg_mm_body,
        grid=(NPAD // _BBLK,),
        in_specs=[
            pl.BlockSpec((1, _BBLK, U), lambda i: (0, i, 0)),
            pl.BlockSpec((1, _BBLK, U), lambda i: (1, i, 0)),
            pl.BlockSpec((2, _BBLK), lambda i: (0, i)),
            pl.BlockSpec((U, U), lambda i: (0, 0)),
            pl.BlockSpec((U, U), lambda i: (0, 0)),
            pl.BlockSpec((U, U), lambda i: (0, 0)),
            pl.BlockSpec((1, U), lambda i: (0, 0)),
        ],
        out_specs=[
            pl.BlockSpec((_BBLK, U), lambda i: (i, 0)),
            pl.BlockSpec((_BBLK, U), lambda i: (i, 0)),
        ],
        out_shape=[
            jax.ShapeDtypeStruct((NPAD, U), jnp.float32),
            jax.ShapeDtypeStruct((NPAD, U), jnp.float32),
        ],
    )(acc, acc, cnt, theta_row, theta_col, theta_all, vsum)


@functools.partial(
    pl.kernel,
    out_type=jax.ShapeDtypeStruct((NNZ, U), jnp.float32),
    mesh=_mesh,
    scratch_types=[
        pltpu.VMEM((2, 128, U), jnp.float32),
        pltpu.VMEM((2, 128, U), jnp.float32),
        pltpu.VMEM((2, 128, U), jnp.float32),
        pltpu.VMEM((81 * 128,), jnp.int32),
        pltpu.VMEM((81 * 128,), jnp.int32),
        pltpu.SemaphoreType.DMA,
        pltpu.SemaphoreType.DMA,
    ],
)
def _gather_finish(self_out, idx2, brow, bcol, out,
                   buf, g1, g2, idxr, idxc, sg0, sg1):
    c = lax.axis_index("c")
    s = lax.axis_index("s")
    w = s * 2 + c
    base = 78 * w + jnp.minimum(w, 4)
    start0 = jnp.minimum(base, NBATCH - 81) * 128
    zero16 = jnp.zeros((16,), jnp.float32)
    sems_g = (sg0, sg1)

    pltpu.sync_copy(idx2.at[0, pl.ds(start0, 81 * 128)], idxr)
    pltpu.sync_copy(idx2.at[1, pl.ds(start0, 81 * 128)], idxc)

    def _b0(j):
        return jnp.minimum(base + j, NBATCH - 1) * 128

    def _issue_loads(j, p):
        b0 = _b0(j)
        pltpu.async_copy(self_out.at[pl.ds(b0, 128), :], buf.at[p], sems_g[p])

    def _issue_gathers(j, p):
        off = _b0(j) - start0
        pltpu.async_copy(brow.at[idxr.at[pl.ds(off, 128)]], g1.at[p],
                         sems_g[p])
        pltpu.async_copy(bcol.at[idxc.at[pl.ds(off, 128)]], g2.at[p],
                         sems_g[p])

    def _compute_store(j, p):
        b0 = _b0(j)
        off = b0 - start0
        pltpu.make_async_copy(self_out.at[pl.ds(b0, 128), :], buf.at[p],
                              sems_g[p]).wait()
        pltpu.make_async_copy(brow.at[idxr.at[pl.ds(off, 128)]], g1.at[p],
                              sems_g[p]).wait()
        pltpu.make_async_copy(bcol.at[idxc.at[pl.ds(off, 128)]], g2.at[p],
                              sems_g[p]).wait()

        def _relu(r, _):
            for k in range(U // 16):
                sl = pl.ds(k * 16, 16)
                v = buf[p, r, sl] + g1[p, r, sl] + g2[p, r, sl]
                buf[p, r, sl] = jnp.maximum(v, zero16)
            return 0
        lax.fori_loop(0, 128, _relu, 0)
        pltpu.sync_copy(buf.at[p], out.at[pl.ds(b0, 128), :])

    _issue_loads(0, 0)
    _issue_gathers(0, 0)

    def _body(t, _):
        j0 = 2 * t
        _issue_loads(j0 + 1, 1)
        _issue_gathers(j0 + 1, 1)
        _compute_store(j0, 0)
        _issue_loads(j0 + 2, 0)
        _issue_gathers(j0 + 2, 0)
        _compute_store(j0 + 1, 1)
        return 0
    lax.fori_loop(0, 40, _body, 0)

    _compute_store(80, 0)


def kernel(values, indices, theta_self, theta_row, theta_col, theta_all):
    idx2 = jnp.transpose(indices.astype(jnp.int32))
    self_out, vsum, cnt_r, cnt_c = _self_mm(values, theta_self, idx2)
    acc = _seg_sums(values, idx2)
    cnt = jnp.stack([cnt_r.reshape(NPAD), cnt_c.reshape(NPAD)])
    brow, bcol = _marg_mm(acc, cnt, theta_row, theta_col, theta_all, vsum)
    return _gather_finish(self_out, idx2, brow, bcol)
